# baseline (speedup 1.0000x reference)
import functools

import jax
import jax.numpy as jnp
from jax import lax
from jax.experimental import pallas as pl
from jax.experimental.pallas import tpu as pltpu
from jax.experimental.pallas import tpu_sc as plsc

_V = 1000000
_D = 64
_B = 4096
_H = 200
_NW = 32
_FULL_COLS = _V // 128
_TAIL = _V - _FULL_COLS * 128
_NCOL_LO = _FULL_COLS // _NW
_NCOL_REM = _FULL_COLS - _NCOL_LO * _NW
_SLOTS = _NCOL_LO + 2
_UNITS = (_H // 8) * (_B // 128)
_UPW = _UNITS // _NW

_mesh = plsc.VectorSubcoreMesh(core_axis_name="c", subcore_axis_name="s")


def _wid():
    return lax.axis_index("s") * 2 + lax.axis_index("c")


def _iota16():
    return lax.iota(jnp.int32, 16)


def _make_reformat():
    @functools.partial(
        pl.kernel,
        mesh=_mesh,
        out_type=jax.ShapeDtypeStruct((_V, 128), jnp.float32),
        scratch_types=[
            pltpu.VMEM((_D, 128), jnp.float32),
            pltpu.VMEM((_D, 128), jnp.float32),
            pltpu.VMEM((128, 128), jnp.float32),
            pltpu.VMEM((128, 128), jnp.float32),
            pltpu.SemaphoreType.DMA,
            pltpu.SemaphoreType.DMA,
            pltpu.SemaphoreType.DMA,
            pltpu.SemaphoreType.DMA,
        ],
        compiler_params=pltpu.CompilerParams(needs_layout_passes=False),
    )
    def reformat(tT_hbm, tail_hbm, scr_hbm, rt0, rt1, w0, w1, r0, r1, s0, s1):
        wid = _wid()
        ncols = _NCOL_LO + jnp.where(wid < _NCOL_REM, 1, 0)
        cbase = wid * _NCOL_LO + jnp.minimum(wid, _NCOL_REM)
        rt = (rt0, rt1)
        w = (w0, w1)
        rsem = (r0, r1)
        ssem = (s0, s1)
        it16 = _iota16()

        def col_off(j):
            return pl.multiple_of((cbase + j) * 128, 128)

        def start_reads(j, b):
            c0 = col_off(j)
            for td in range(8):
                pltpu.async_copy(
                    tT_hbm.at[pl.ds(td * 8, 8), pl.ds(c0, 128)],
                    rt[b].at[pl.ds(td * 8, 8)],
                    rsem[b],
                )

        def wait_reads(b):
            for td in range(8):
                pltpu.make_async_copy(
                    tT_hbm.at[pl.ds(0, 8), pl.ds(0, 128)],
                    rt[b].at[pl.ds(td * 8, 8)],
                    rsem[b],
                ).wait()

        def start_write(j, b):
            pltpu.async_copy(
                w[b],
                scr_hbm.at[pl.ds(col_off(j), 128)],
                ssem[b],
            )

        def wait_write(b):
            pltpu.make_async_copy(
                w[b],
                scr_hbm.at[pl.ds(0, 128)],
                ssem[b],
            ).wait()

        dgv = [dg * 16 + it16 for dg in range(4)]

        def transpose_block(b, nv):
            @plsc.parallel_loop(0, nv, unroll=16)
            def tv(v):
                cols = jnp.full((16,), 0, jnp.int32) + v
                for dg in range(4):
                    vec = plsc.load_gather(rt[b], [dgv[dg], cols])
                    plsc.store_scatter(w[b], [cols, dgv[dg]], vec)

        start_reads(0, 0)

        def body(i, carry):
            for bb in range(2):
                j = 2 * i + bb
                b = bb

                @pl.when(j < ncols)
                def _():
                    wait_reads(b)

                @pl.when(j + 1 < ncols)
                def _():
                    start_reads(j + 1, 1 - b)

                @pl.when((j >= 2) & (j - 2 < ncols))
                def _():
                    wait_write(b)

                @pl.when(j < ncols)
                def _():
                    transpose_block(b, 128)
                    start_write(j, b)

            return carry

        lax.fori_loop(0, _SLOTS // 2, body, 0)

        @pl.when(ncols == _NCOL_LO + 1)
        def _():
            wait_write((_NCOL_LO + 1 - 1) % 2)

        @pl.when(wid == 0)
        def _():
            pltpu.sync_copy(tail_hbm, w0.at[pl.ds(0, _TAIL)])
            pltpu.async_copy(
                w0.at[pl.ds(0, _TAIL)],
                scr_hbm.at[pl.ds(_FULL_COLS * 128, _TAIL)],
                s0,
            ).wait()

    return reformat


def _make_gather():
    @functools.partial(
        pl.kernel,
        mesh=_mesh,
        out_type=jax.ShapeDtypeStruct((_H, _D, _B), jnp.float32),
        scratch_types=[
            pltpu.VMEM((8, 128), jnp.int32),
            pltpu.VMEM((8, 128), jnp.int32),
            pltpu.VMEM((128,), jnp.int32),
            pltpu.VMEM((128,), jnp.int32),
            pltpu.VMEM((128, 128), jnp.float32),
            pltpu.VMEM((128, 128), jnp.float32),
            pltpu.VMEM((_D, 128), jnp.float32),
            pltpu.VMEM((_D, 128), jnp.float32),
            pltpu.SemaphoreType.DMA,
            pltpu.SemaphoreType.DMA,
            pltpu.SemaphoreType.DMA,
            pltpu.SemaphoreType.DMA,
            pltpu.SemaphoreType.DMA,
            pltpu.SemaphoreType.DMA,
        ],
        compiler_params=pltpu.CompilerParams(needs_layout_passes=False),
    )
    def gather(
        xT_hbm, scr_hbm, outT_hbm,
        xi0, xi1, ix0, ix1, rr0, rr1, ww0, ww1,
        xs0, xs1, g0, g1, s0, s1,
    ):
        wid = _wid()
        xi = (xi0, xi1)
        ix = (ix0, ix1)
        rr = (rr0, rr1)
        ww = (ww0, ww1)
        xsem = (xs0, xs1)
        gsem = (g0, g1)
        ssem = (s0, s1)
        it16 = _iota16()

        def unit_ht(u):
            return u // (_B // 128)

        def unit_bt(u):
            return u % (_B // 128)

        def start_xi(u, b):
            ht = unit_ht(u)
            bt = unit_bt(u)
            pltpu.async_copy(
                xT_hbm.at[
                    pl.ds(pl.multiple_of(ht * 8, 8), 8),
                    pl.ds(pl.multiple_of(bt * 128, 128), 128),
                ],
                xi[b],
                xsem[b],
            )

        def wait_xi(b):
            pltpu.make_async_copy(
                xT_hbm.at[pl.ds(0, 8), pl.ds(0, 128)], xi[b], xsem[b]
            ).wait()

        def extract_row(xb, hs, b):
            for g in range(8):
                ix[b][pl.ds(g * 16, 16)] = xi[xb][hs, pl.ds(g * 16, 16)]

        def start_gather(b):
            pltpu.async_copy(scr_hbm.at[ix[b]], rr[b], gsem[b])

        def wait_gather(b):
            pltpu.make_async_copy(
                scr_hbm.at[ix[b]], rr[b], gsem[b]
            ).wait()

        bgv = [bg * 16 + it16 for bg in range(8)]

        def transpose_block(b):
            @plsc.parallel_loop(0, _D, unroll=16)
            def td(d):
                cols = jnp.full((16,), 0, jnp.int32) + d
                for bg in range(8):
                    vec = plsc.load_gather(rr[b], [bgv[bg], cols])
                    plsc.store_scatter(ww[b], [cols, bgv[bg]], vec)

        def start_store(u, hs, b):
            h = unit_ht(u) * 8 + hs
            bt = unit_bt(u)
            pltpu.async_copy(
                ww[b],
                outT_hbm.at[
                    h, pl.ds(0, _D), pl.ds(pl.multiple_of(bt * 128, 128), 128)
                ],
                ssem[b],
            )

        def wait_store(b):
            pltpu.make_async_copy(
                ww[b],
                outT_hbm.at[0, pl.ds(0, _D), pl.ds(0, 128)],
                ssem[b],
            ).wait()

        ubase = wid * _UPW
        start_xi(ubase, 0)

        def unit_body(i, carry):
            for kb in range(2):
                k = 2 * i + kb
                u = ubase + k
                xb = kb

                @pl.when(k < _UPW)
                def _():
                    wait_xi(xb)

                    @pl.when(k + 1 < _UPW)
                    def _():
                        start_xi(u + 1, 1 - xb)

                    extract_row(xb, 0, 0)
                    start_gather(0)
                    for hs in range(8):
                        b = hs % 2
                        wait_gather(b)
                        if hs + 1 < 8:
                            extract_row(xb, hs + 1, 1 - b)
                            start_gather(1 - b)
                        wait_store_maybe(k, hs, b)
                        transpose_block(b)
                        start_store(u, hs, b)

            return carry

        def wait_store_maybe(k, hs, b):
            first = (k == 0) & (hs < 2)

            @pl.when(jnp.logical_not(first))
            def _():
                wait_store(b)

        lax.fori_loop(0, (_UPW + 1) // 2, unit_body, 0)
        wait_store(0)
        wait_store(1)

    return gather


_reformat = _make_reformat()
_gather = _make_gather()


def kernel(x, table):
    tail = jnp.pad(table[_FULL_COLS * 128 :, :], ((0, 0), (0, 128 - _D)))
    scr = _reformat(table.T, tail)
    outT = _gather(x.T.astype(jnp.int32), scr)
    return outT.transpose(2, 0, 1)


# speedup vs baseline: 1.2224x; 1.2224x over previous
import functools

import jax
import jax.numpy as jnp
from jax import lax
from jax.experimental import pallas as pl
from jax.experimental.pallas import tpu as pltpu
from jax.experimental.pallas import tpu_sc as plsc

_VOCAB = 1000000
_D = 64
_B = 4096
_H = 200
_TOT = _B * _H
_NW = 32
_PER_W = _TOT // _NW
_C = 640
_NCH = _PER_W // _C


def _make_kernel():
    mesh = plsc.VectorSubcoreMesh(core_axis_name="c", subcore_axis_name="s")

    @functools.partial(
        pl.kernel,
        mesh=mesh,
        out_type=jax.ShapeDtypeStruct((_TOT, _D), jnp.float32),
        scratch_types=[
            pltpu.VMEM((_PER_W,), jnp.int32),
            pltpu.VMEM((_C, _D), jnp.float32),
            pltpu.VMEM((_C, _D), jnp.float32),
            pltpu.SemaphoreType.DMA,
            pltpu.SemaphoreType.DMA,
            pltpu.SemaphoreType.DMA,
            pltpu.SemaphoreType.DMA,
        ],
        compiler_params=pltpu.CompilerParams(use_tc_tiling_on_sc=False),
    )
    def emb(x_hbm, table_hbm, out_hbm, idx_all, rows0, rows1, g0, g1, s0, s1):
        wid = lax.axis_index("s") * 2 + lax.axis_index("c")
        base = wid * _PER_W
        pltpu.sync_copy(x_hbm.at[wid], idx_all)

        rows = (rows0, rows1)
        gsem = (g0, g1)
        ssem = (s0, s1)

        def idx_slice(j):
            return idx_all.at[pl.ds(pl.multiple_of(j * _C, _C), _C)]

        def start_gather(j, b):
            pltpu.async_copy(table_hbm.at[idx_slice(j)], rows[b], gsem[b])

        def start_store(j, b):
            pltpu.async_copy(
                rows[b], out_hbm.at[pl.ds(base + j * _C, _C)], ssem[b]
            )

        def wait_gather(b):
            pltpu.make_async_copy(
                table_hbm.at[idx_slice(0)], rows[b], gsem[b]
            ).wait()

        def wait_store(b):
            pltpu.make_async_copy(
                rows[b], out_hbm.at[pl.ds(base, _C)], ssem[b]
            ).wait()

        start_gather(0, 0)

        def body(i, carry):
            for b in (0, 1):
                j = 2 * i + b
                nb = 1 - b
                wait_gather(b)

                @pl.when(j >= 1)
                def _():
                    wait_store(nb)

                @pl.when(j + 1 < _NCH)
                def _():
                    start_gather(j + 1, nb)

                start_store(j, b)
            return carry

        lax.fori_loop(0, _NCH // 2, body, 0)
        wait_store((_NCH - 1) % 2)

    return emb


_emb = _make_kernel()


def kernel(x, table):
    xf = x.reshape(_NW, _PER_W).astype(jnp.int32)
    out = _emb(xf, table)
    return out.reshape(_B, _H, _D)


# speedup vs baseline: 1.4895x; 1.2185x over previous
import functools

import jax
import jax.numpy as jnp
from jax import lax
from jax.experimental import pallas as pl
from jax.experimental.pallas import tpu as pltpu
from jax.experimental.pallas import tpu_sc as plsc

_VOCAB = 1000000
_D = 64
_DP = 128
_B = 4096
_H = 200
_TOT = _B * _H
_NW = 32
_PER_W = _TOT // _NW
_C = 256
_NCH = _PER_W // _C


def _make_kernel():
    mesh = plsc.VectorSubcoreMesh(core_axis_name="c", subcore_axis_name="s")

    @functools.partial(
        pl.kernel,
        mesh=mesh,
        out_type=jax.ShapeDtypeStruct((_TOT, _DP), jnp.float32),
        scratch_types=[
            pltpu.VMEM((_PER_W,), jnp.int32),
            pltpu.VMEM((_C, _DP), jnp.float32),
            pltpu.VMEM((_C, _DP), jnp.float32),
            pltpu.SemaphoreType.DMA,
            pltpu.SemaphoreType.DMA,
            pltpu.SemaphoreType.DMA,
            pltpu.SemaphoreType.DMA,
        ],
        compiler_params=pltpu.CompilerParams(use_tc_tiling_on_sc=False),
    )
    def emb(x_hbm, table_hbm, out_hbm, idx_all, rows0, rows1, g0, g1, s0, s1):
        wid = lax.axis_index("s") * 2 + lax.axis_index("c")
        base = wid * _PER_W
        pltpu.sync_copy(x_hbm.at[wid], idx_all)

        rows = (rows0, rows1)
        gsem = (g0, g1)
        ssem = (s0, s1)

        def idx_slice(j):
            return idx_all.at[pl.ds(pl.multiple_of(j * _C, _C), _C)]

        def start_gather(j, b):
            pltpu.async_copy(table_hbm.at[idx_slice(j)], rows[b], gsem[b])

        def start_store(j, b):
            pltpu.async_copy(
                rows[b], out_hbm.at[pl.ds(base + j * _C, _C)], ssem[b]
            )

        def wait_gather(b):
            pltpu.make_async_copy(
                table_hbm.at[idx_slice(0)], rows[b], gsem[b]
            ).wait()

        def wait_store(b):
            pltpu.make_async_copy(
                rows[b], out_hbm.at[pl.ds(base, _C)], ssem[b]
            ).wait()

        start_gather(0, 0)

        def body(i, carry):
            for b in (0, 1):
                j = 2 * i + b
                nb = 1 - b
                wait_gather(b)

                @pl.when(j >= 1)
                def _():
                    wait_store(nb)

                @pl.when(j + 1 < _NCH)
                def _():
                    start_gather(j + 1, nb)

                start_store(j, b)
            return carry

        lax.fori_loop(0, _NCH // 2, body, 0)
        wait_store((_NCH - 1) % 2)

    return emb


_emb = _make_kernel()


def kernel(x, table):
    xf = x.reshape(_NW, _PER_W).astype(jnp.int32)
    table_p = jnp.pad(table, ((0, 0), (0, _DP - _D)))
    out = _emb(xf, table_p)
    return out.reshape(_B, _H, _DP)[:, :, :_D]
